# baseline (speedup 1.0000x reference)
import functools

import jax
import jax.numpy as jnp
from jax import lax
from jax.experimental import pallas as pl
from jax.experimental.pallas import tpu as pltpu
from jax.experimental.pallas import tpu_sc as plsc


def _build_gather(B: int, D: int):
    info = plsc.get_sparse_core_info()
    NC, NS, L = info.num_cores, info.num_subcores, info.num_lanes
    NW = NC * NS
    assert B % (8 * NW) == 0 and D % L == 0
    b_per_w = B // NW
    CHUNK = 800
    assert b_per_w % CHUNK == 0
    NCHUNK = b_per_w // CHUNK

    mesh = plsc.VectorSubcoreMesh(core_axis_name="c", subcore_axis_name="s")

    @functools.partial(
        pl.kernel,
        mesh=mesh,
        compiler_params=pltpu.CompilerParams(use_tc_tiling_on_sc=False),
        out_type=jax.ShapeDtypeStruct((B, D), jnp.float32),
        scratch_types=[
            pltpu.VMEM((b_per_w,), jnp.int32),
            pltpu.VMEM((CHUNK, D), jnp.float32),
            pltpu.SemaphoreType.DMA,
        ],
    )
    def gather2x(table_hbm, idx_hbm, out_hbm, idx_v, rows_v, sem):
        wid = lax.axis_index("s") * NC + lax.axis_index("c")
        base = wid * b_per_w
        pltpu.sync_copy(idx_hbm.at[pl.ds(base, b_per_w)], idx_v)

        def chunk_body(j, carry):
            cb = j * CHUNK
            pltpu.async_copy(
                table_hbm.at[idx_v.at[pl.ds(cb, CHUNK)]], rows_v, sem
            ).wait()

            def mul_body(r, c2):
                for c in range(D // L):
                    sl = pl.ds(c * L, L)
                    rows_v[r, sl] = rows_v[r, sl] * 2.0
                return c2

            lax.fori_loop(0, CHUNK, mul_body, 0, unroll=4)
            pltpu.sync_copy(rows_v, out_hbm.at[pl.ds(base + cb, CHUNK)])
            return carry

        lax.fori_loop(0, NCHUNK, chunk_body, 0)

    return gather2x


def kernel(inputs, token_table, position_table):
    del position_table
    Bx, S = inputs.shape
    V, D = token_table.shape
    idx = inputs.reshape(-1).astype(jnp.int32)
    out = _build_gather(Bx * S, D)(token_table, idx)
    return out.reshape(Bx, S, D)
